# baseline (speedup 1.0000x reference)
import functools

import jax
import jax.numpy as jnp
from jax import lax
from jax.experimental import pallas as pl
from jax.experimental.pallas import tpu as pltpu
from jax.experimental.pallas import tpu_sc as plsc

BATCH = 16384
N_FEATURES = 100
D_MODEL = 32
N_CLASSES = 1000000

NUM_CORES = 2
NUM_SUBCORES = 16
NUM_WORKERS = NUM_CORES * NUM_SUBCORES

NUM_SLICES = 4
BS = BATCH // NUM_SLICES
B_PER_WORKER = BS // NUM_WORKERS
B_PER_CHUNK = 16
NUM_CHUNKS = B_PER_WORKER // B_PER_CHUNK
Z_ROWS = BS * N_FEATURES * D_MODEL // 128


def _gather_body(s_off, x_hbm, emb_hbm, z_hbm,
                 idx0, idx1, rows0, rows1, gsem0, gsem1, wsem0, wsem1):
    wid = lax.axis_index("s") * NUM_CORES + lax.axis_index("c")
    b_local = wid * B_PER_WORKER
    b_global = s_off + b_local
    idx = (idx0, idx1)
    rows = (rows0, rows1)
    gsem = (gsem0, gsem1)
    wsem = (wsem0, wsem1)

    def fire_gathers(g, buf):
        cb = b_global + g * B_PER_CHUNK
        pltpu.sync_copy(x_hbm.at[pl.ds(cb, B_PER_CHUNK)], idx[buf])
        for i in range(B_PER_CHUNK):
            pltpu.make_async_copy(
                emb_hbm.at[idx[buf].at[i]], rows[buf].at[i], gsem[buf]
            ).start()

    def wait_gathers(buf):
        for i in range(B_PER_CHUNK):
            pltpu.make_async_copy(
                emb_hbm.at[idx[buf].at[i]], rows[buf].at[i], gsem[buf]
            ).wait()

    def start_writes(g, buf):
        cb = b_local + g * B_PER_CHUNK
        b512 = cb // 512
        q = (cb % 512) // 128
        k0 = cb % 128

        def w_body(f, carry):
            row0 = (f * (BS // 512) + b512) * 128 + k0
            pltpu.make_async_copy(
                rows[buf].at[:, f, :],
                z_hbm.at[pl.ds(row0, B_PER_CHUNK), pl.ds(q * D_MODEL, D_MODEL)],
                wsem[buf],
            ).start()
            return carry

        lax.fori_loop(0, N_FEATURES, w_body, 0, unroll=False)

    def drain_writes(buf):
        def d_body(f, carry):
            pltpu.make_async_copy(
                rows[buf].at[:, 0, :],
                z_hbm.at[pl.ds(0, B_PER_CHUNK), pl.ds(0, D_MODEL)],
                wsem[buf],
            ).wait()
            return carry

        lax.fori_loop(0, N_FEATURES, d_body, 0, unroll=False)

    fire_gathers(0, 0)

    def pair_body(p, carry):
        for b in (0, 1):
            g = 2 * p + b
            wait_gathers(b)
            start_writes(g, b)

            @pl.when(g < NUM_CHUNKS - 1)
            def _():
                @pl.when(g > 0)
                def _():
                    drain_writes(1 - b)
                fire_gathers(g + 1, 1 - b)

        return carry

    lax.fori_loop(0, NUM_CHUNKS // 2, pair_body, 0, unroll=False)
    drain_writes(0)
    drain_writes(1)


@functools.partial(jax.jit, static_argnums=(2,))
def _sc_gather(x, embeddings, s_off):
    mesh = plsc.VectorSubcoreMesh(
        core_axis_name="c", subcore_axis_name="s",
        num_cores=NUM_CORES, num_subcores=NUM_SUBCORES,
    )
    return pl.kernel(
        functools.partial(_gather_body, s_off),
        out_type=jax.ShapeDtypeStruct((Z_ROWS, 128), jnp.float32),
        name=f"sc_gather_{s_off}",
        mesh=mesh,
        compiler_params=pltpu.CompilerParams(use_tc_tiling_on_sc=False),
        scratch_types=[
            pltpu.VMEM((B_PER_CHUNK, N_FEATURES), jnp.int32),
            pltpu.VMEM((B_PER_CHUNK, N_FEATURES), jnp.int32),
            pltpu.VMEM((B_PER_CHUNK, N_FEATURES, D_MODEL), jnp.float32),
            pltpu.VMEM((B_PER_CHUNK, N_FEATURES, D_MODEL), jnp.float32),
            pltpu.SemaphoreType.DMA,
            pltpu.SemaphoreType.DMA,
            pltpu.SemaphoreType.DMA,
            pltpu.SemaphoreType.DMA,
        ],
    )(x, embeddings)


def _trans_body(z_ref, ident_ref, fe_ref, out_ref):
    z3 = z_ref[...].reshape(BS // 512, 128, 128)
    xt = jax.lax.dot_general(
        z3, ident_ref[...], (((1,), (0,)), ((), ())),
        preferred_element_type=jnp.float32,
        precision=jax.lax.Precision.HIGHEST)
    out2 = jnp.transpose(xt.reshape(BS // 512, 4, D_MODEL, 128), (2, 0, 1, 3))
    out2 = out2.reshape(D_MODEL, BS)
    bias = fe_ref[pl.program_id(0), :]
    out_ref[...] = (out2 + bias[:, None])[None]


def _trans_body_alias(z_ref, ident_ref, fe_ref, prev_ref, out_ref):
    del prev_ref
    _trans_body(z_ref, ident_ref, fe_ref, out_ref)


@jax.jit
def _tc_transpose0(z, feature_emb):
    return pl.pallas_call(
        _trans_body,
        grid=(N_FEATURES,),
        in_specs=[
            pl.BlockSpec((BS * D_MODEL // 128, 128), lambda f: (f, 0)),
            pl.BlockSpec((128, 128), lambda f: (0, 0)),
            pl.BlockSpec((N_FEATURES, D_MODEL), lambda f: (0, 0)),
        ],
        out_specs=pl.BlockSpec((1, D_MODEL, BS), lambda f: (f, 0, 0)),
        out_shape=jax.ShapeDtypeStruct((N_FEATURES, D_MODEL, BATCH), jnp.float32),
    )(z, jnp.eye(128, dtype=jnp.float32), feature_emb)


@functools.partial(jax.jit, static_argnums=(3,))
def _tc_transpose_s(z, feature_emb, prev, s):
    return pl.pallas_call(
        _trans_body_alias,
        grid=(N_FEATURES,),
        in_specs=[
            pl.BlockSpec((BS * D_MODEL // 128, 128), lambda f: (f, 0)),
            pl.BlockSpec((128, 128), lambda f: (0, 0)),
            pl.BlockSpec((N_FEATURES, D_MODEL), lambda f: (0, 0)),
            pl.BlockSpec(memory_space=pl.ANY),
        ],
        out_specs=pl.BlockSpec((1, D_MODEL, BS), lambda f: (f, 0, s)),
        out_shape=jax.ShapeDtypeStruct((N_FEATURES, D_MODEL, BATCH), jnp.float32),
        input_output_aliases={3: 0},
    )(z, jnp.eye(128, dtype=jnp.float32), feature_emb, prev)


def kernel(x, embeddings, feature_emb):
    xb = lax.optimization_barrier(
        jnp.reshape(jnp.asarray(x, jnp.int32), (BATCH * N_FEATURES // 128, 128)))
    x2 = jnp.reshape(xb, (BATCH, N_FEATURES))
    tb = lax.optimization_barrier(
        jnp.reshape(embeddings, (N_CLASSES * D_MODEL // 128, 128)))
    table = jnp.reshape(tb, (N_CLASSES, D_MODEL))

    z = _sc_gather(x2, table, 0)
    z_next = _sc_gather(x2, table, BS)
    outT = _tc_transpose0(z, feature_emb)
    for s in range(1, NUM_SLICES):
        z = z_next
        if s + 1 < NUM_SLICES:
            z_next = _sc_gather(x2, table, (s + 1) * BS)
        outT = _tc_transpose_s(z, feature_emb, outT, s)
    return jnp.transpose(outT, (2, 0, 1))


# speedup vs baseline: 1.0813x; 1.0813x over previous
import functools

import jax
import jax.numpy as jnp
from jax import lax
from jax.experimental import pallas as pl
from jax.experimental.pallas import tpu as pltpu
from jax.experimental.pallas import tpu_sc as plsc

BATCH = 16384
N_FEATURES = 100
D_MODEL = 32
N_CLASSES = 1000000

NUM_CORES = 2
NUM_SUBCORES = 16
NUM_WORKERS = NUM_CORES * NUM_SUBCORES

NUM_SLICES = 2
BS = BATCH // NUM_SLICES
B_PER_WORKER = BS // NUM_WORKERS
B_PER_CHUNK = 16
NUM_CHUNKS = B_PER_WORKER // B_PER_CHUNK
Z_ROWS = BS * N_FEATURES * D_MODEL // 128


def _gather_body(s_off, x_hbm, emb_hbm, z_hbm,
                 idx0, idx1, rows0, rows1, gsem0, gsem1, wsem0, wsem1):
    wid = lax.axis_index("s") * NUM_CORES + lax.axis_index("c")
    b_local = wid * B_PER_WORKER
    b_global = s_off + b_local
    idx = (idx0, idx1)
    rows = (rows0, rows1)
    gsem = (gsem0, gsem1)
    wsem = (wsem0, wsem1)

    def fire_gathers(g, buf):
        cb = b_global + g * B_PER_CHUNK
        pltpu.sync_copy(x_hbm.at[pl.ds(cb, B_PER_CHUNK)], idx[buf])
        for i in range(B_PER_CHUNK):
            pltpu.make_async_copy(
                emb_hbm.at[idx[buf].at[i]], rows[buf].at[i], gsem[buf]
            ).start()

    def wait_gathers(buf):
        for i in range(B_PER_CHUNK):
            pltpu.make_async_copy(
                emb_hbm.at[idx[buf].at[i]], rows[buf].at[i], gsem[buf]
            ).wait()

    def start_writes(g, buf):
        cb = b_local + g * B_PER_CHUNK
        b512 = cb // 512
        q = (cb % 512) // 128
        k0 = cb % 128

        def w_body(f, carry):
            row0 = (f * (BS // 512) + b512) * 128 + k0
            pltpu.make_async_copy(
                rows[buf].at[:, f, :],
                z_hbm.at[pl.ds(row0, B_PER_CHUNK), pl.ds(q * D_MODEL, D_MODEL)],
                wsem[buf],
            ).start()
            return carry

        lax.fori_loop(0, N_FEATURES, w_body, 0, unroll=False)

    def drain_writes(buf):
        def d_body(f, carry):
            pltpu.make_async_copy(
                rows[buf].at[:, 0, :],
                z_hbm.at[pl.ds(0, B_PER_CHUNK), pl.ds(0, D_MODEL)],
                wsem[buf],
            ).wait()
            return carry

        lax.fori_loop(0, N_FEATURES, d_body, 0, unroll=False)

    fire_gathers(0, 0)

    def pair_body(p, carry):
        for b in (0, 1):
            g = 2 * p + b
            wait_gathers(b)
            start_writes(g, b)

            @pl.when(g < NUM_CHUNKS - 1)
            def _():
                @pl.when(g > 0)
                def _():
                    drain_writes(1 - b)
                fire_gathers(g + 1, 1 - b)

        return carry

    lax.fori_loop(0, NUM_CHUNKS // 2, pair_body, 0, unroll=False)
    drain_writes(0)
    drain_writes(1)


@functools.partial(jax.jit, static_argnums=(2,))
def _sc_gather(x, embeddings, s_off):
    mesh = plsc.VectorSubcoreMesh(
        core_axis_name="c", subcore_axis_name="s",
        num_cores=NUM_CORES, num_subcores=NUM_SUBCORES,
    )
    return pl.kernel(
        functools.partial(_gather_body, s_off),
        out_type=jax.ShapeDtypeStruct((Z_ROWS, 128), jnp.float32),
        name=f"sc_gather_{s_off}",
        mesh=mesh,
        compiler_params=pltpu.CompilerParams(use_tc_tiling_on_sc=False),
        scratch_types=[
            pltpu.VMEM((B_PER_CHUNK, N_FEATURES), jnp.int32),
            pltpu.VMEM((B_PER_CHUNK, N_FEATURES), jnp.int32),
            pltpu.VMEM((B_PER_CHUNK, N_FEATURES, D_MODEL), jnp.float32),
            pltpu.VMEM((B_PER_CHUNK, N_FEATURES, D_MODEL), jnp.float32),
            pltpu.SemaphoreType.DMA,
            pltpu.SemaphoreType.DMA,
            pltpu.SemaphoreType.DMA,
            pltpu.SemaphoreType.DMA,
        ],
    )(x, embeddings)


def _trans_body(z_ref, ident_ref, fe_ref, out_ref):
    z3 = z_ref[...].reshape(BS // 512, 128, 128)
    xt = jax.lax.dot_general(
        z3, ident_ref[...], (((1,), (0,)), ((), ())),
        preferred_element_type=jnp.float32,
        precision=jax.lax.Precision.HIGHEST)
    out2 = jnp.transpose(xt.reshape(BS // 512, 4, D_MODEL, 128), (2, 0, 1, 3))
    out2 = out2.reshape(D_MODEL, BS)
    bias = fe_ref[pl.program_id(0), :]
    out_ref[...] = (out2 + bias[:, None])[None]


def _trans_body_alias(z_ref, ident_ref, fe_ref, prev_ref, out_ref):
    del prev_ref
    _trans_body(z_ref, ident_ref, fe_ref, out_ref)


@jax.jit
def _tc_transpose0(z, feature_emb):
    return pl.pallas_call(
        _trans_body,
        grid=(N_FEATURES,),
        in_specs=[
            pl.BlockSpec((BS * D_MODEL // 128, 128), lambda f: (f, 0)),
            pl.BlockSpec((128, 128), lambda f: (0, 0)),
            pl.BlockSpec((N_FEATURES, D_MODEL), lambda f: (0, 0)),
        ],
        out_specs=pl.BlockSpec((1, D_MODEL, BS), lambda f: (f, 0, 0)),
        out_shape=jax.ShapeDtypeStruct((N_FEATURES, D_MODEL, BATCH), jnp.float32),
    )(z, jnp.eye(128, dtype=jnp.float32), feature_emb)


@functools.partial(jax.jit, static_argnums=(3,))
def _tc_transpose_s(z, feature_emb, prev, s):
    return pl.pallas_call(
        _trans_body_alias,
        grid=(N_FEATURES,),
        in_specs=[
            pl.BlockSpec((BS * D_MODEL // 128, 128), lambda f: (f, 0)),
            pl.BlockSpec((128, 128), lambda f: (0, 0)),
            pl.BlockSpec((N_FEATURES, D_MODEL), lambda f: (0, 0)),
            pl.BlockSpec(memory_space=pl.ANY),
        ],
        out_specs=pl.BlockSpec((1, D_MODEL, BS), lambda f: (f, 0, s)),
        out_shape=jax.ShapeDtypeStruct((N_FEATURES, D_MODEL, BATCH), jnp.float32),
        input_output_aliases={3: 0},
    )(z, jnp.eye(128, dtype=jnp.float32), feature_emb, prev)


def kernel(x, embeddings, feature_emb):
    xb = lax.optimization_barrier(
        jnp.reshape(jnp.asarray(x, jnp.int32), (BATCH * N_FEATURES // 128, 128)))
    x2 = jnp.reshape(xb, (BATCH, N_FEATURES))
    tb = lax.optimization_barrier(
        jnp.reshape(embeddings, (N_CLASSES * D_MODEL // 128, 128)))
    table = jnp.reshape(tb, (N_CLASSES, D_MODEL))

    z = _sc_gather(x2, table, 0)
    z_next = _sc_gather(x2, table, BS)
    outT = _tc_transpose0(z, feature_emb)
    for s in range(1, NUM_SLICES):
        z = z_next
        if s + 1 < NUM_SLICES:
            z_next = _sc_gather(x2, table, (s + 1) * BS)
        outT = _tc_transpose_s(z, feature_emb, outT, s)
    return jnp.transpose(outT, (2, 0, 1))
